# Initial kernel scaffold; baseline (speedup 1.0000x reference)
import jax
import jax.numpy as jnp
from jax.experimental import pallas as pl


def kernel(x, edge_index, batch, W1, b1, W2, b2, Wfc, bfc):
    raise NotImplementedError("write your pallas kernel here")


# speedup vs baseline: 6.3608x; 6.3608x over previous
import functools

import jax
import jax.numpy as jnp
from jax import lax
from jax.experimental import pallas as pl
from jax.experimental.pallas import tpu as pltpu
from jax.experimental.pallas import tpu_sc as plsc

_N = 10000
_E = 320000
_F = 128
_H = 16
_C = 10
_G = 64

_NC = 2
_NS = 16
_NW = _NC * _NS
_EPW = _E // _NW
_CH = 80
_NCHUNK = _EPW // _CH
_NP = 10240
_RPT = _NP // _NS

_sc_mesh = plsc.VectorSubcoreMesh(core_axis_name="c", subcore_axis_name="s")


@functools.partial(
    pl.kernel,
    mesh=_sc_mesh,
    compiler_params=pltpu.CompilerParams(use_tc_tiling_on_sc=False),
    out_type=jax.ShapeDtypeStruct((_NC * _NP, _H), jnp.float32),
    scratch_types=[
        pltpu.VMEM((_CH,), jnp.int32),
        pltpu.VMEM((_CH,), jnp.int32),
        pltpu.VMEM((_CH, _H), jnp.float32),
        pltpu.VMEM((_RPT, _H), jnp.float32),
        pltpu.VMEM_SHARED((_NP, _H), jnp.float32),
        pltpu.SemaphoreType.DMA,
    ],
)
def _edge_segsum(y_hbm, src_hbm, dst_hbm, out_hbm, src_v, dst_v, rows_v, buf_v,
                 acc_sh, sem):
    cid = lax.axis_index("c")
    sid = lax.axis_index("s")
    wid = sid * _NC + cid

    def _zero_row(i, carry):
        buf_v[i] = jnp.zeros((_H,), jnp.float32)
        return carry

    lax.fori_loop(0, _RPT, _zero_row, 0)
    pltpu.sync_copy(buf_v, acc_sh.at[pl.ds(sid * _RPT, _RPT)])
    plsc.subcore_barrier()

    base0 = wid * _EPW

    def _chunk(c, carry):
        base = base0 + c * _CH
        pltpu.sync_copy(src_hbm.at[pl.ds(base, _CH)], src_v)
        pltpu.sync_copy(dst_hbm.at[pl.ds(base, _CH)], dst_v)
        pltpu.async_copy(y_hbm.at[src_v], rows_v, sem).wait()
        pltpu.sync_copy(rows_v, acc_sh.at[dst_v], add=True)
        return carry

    lax.fori_loop(0, _NCHUNK, _chunk, 0)
    plsc.subcore_barrier()

    r0 = sid * _RPT
    pltpu.sync_copy(acc_sh.at[pl.ds(r0, _RPT)], buf_v)
    pltpu.sync_copy(buf_v, out_hbm.at[pl.ds(cid * _NP + r0, _RPT)])


def _mm1_body(x_ref, w_ref, o_ref):
    o_ref[...] = jnp.dot(x_ref[...], w_ref[...], preferred_element_type=jnp.float32)


_mm1 = pl.pallas_call(
    _mm1_body,
    out_shape=jax.ShapeDtypeStruct((_N, _H), jnp.float32),
)


def _mid_body(y_ref, s_ref, b1_ref, w2_ref, z_ref):
    h = y_ref[...] + s_ref[: _N, :] + s_ref[_NP : _NP + _N, :] + b1_ref[...]
    h = jnp.maximum(h, 0.0)
    z_ref[...] = jnp.dot(h, w2_ref[...], preferred_element_type=jnp.float32)


_mid = pl.pallas_call(
    _mid_body,
    out_shape=jax.ShapeDtypeStruct((_N, _H), jnp.float32),
)


def _final_body(z_ref, s_ref, b2_ref, batch_ref, wfc_ref, bfc_ref, o_ref):
    h2 = z_ref[...] + s_ref[: _N, :] + s_ref[_NP : _NP + _N, :] + b2_ref[...]
    h2 = jnp.maximum(h2, 0.0)
    gid = lax.broadcasted_iota(jnp.int32, (_G, _N), 0)
    mask = jnp.where(batch_ref[...] == gid, 1.0, 0.0)
    sums = jnp.dot(mask, h2, preferred_element_type=jnp.float32)
    counts = jnp.sum(mask, axis=1, keepdims=True)
    pooled = sums / jnp.maximum(counts, 1.0)
    o_ref[...] = (
        jnp.dot(pooled, wfc_ref[...], preferred_element_type=jnp.float32)
        + bfc_ref[...]
    )


_final = pl.pallas_call(
    _final_body,
    out_shape=jax.ShapeDtypeStruct((_G, _C), jnp.float32),
)


def kernel(x, edge_index, batch, W1, b1, W2, b2, Wfc, bfc):
    src = edge_index[0]
    dst = edge_index[1]
    y = _mm1(x, W1)
    s1 = _edge_segsum(y, src, dst)
    z = _mid(y, s1, b1.reshape(1, _H), W2)
    s2 = _edge_segsum(z, src, dst)
    return _final(z, s2, b2.reshape(1, _H), batch.reshape(1, _N), Wfc,
                  bfc.reshape(1, _C))


# speedup vs baseline: 22.3316x; 3.5108x over previous
import functools

import jax
import jax.numpy as jnp
from jax import lax
from jax.experimental import pallas as pl
from jax.experimental.pallas import tpu as pltpu
from jax.experimental.pallas import tpu_sc as plsc

_N = 10000
_E = 320000
_F = 128
_H = 16
_C = 10
_G = 64

_NC = 2
_NS = 16
_NW = _NC * _NS
_EPW = _E // _NW
_CH = 80
_NCHUNK = _EPW // _CH
_NBUF = 5
_NP = 10240
_RPT = _NP // _NS

_sc_mesh = plsc.VectorSubcoreMesh(core_axis_name="c", subcore_axis_name="s")


@functools.partial(
    pl.kernel,
    mesh=_sc_mesh,
    compiler_params=pltpu.CompilerParams(use_tc_tiling_on_sc=False),
    out_type=jax.ShapeDtypeStruct((_NC * _NP, _H), jnp.float32),
    scratch_types=[
        pltpu.VMEM((_NCHUNK, _CH), jnp.int32),
        pltpu.VMEM((_NCHUNK, _CH), jnp.int32),
        pltpu.VMEM((_NBUF, _CH, _H), jnp.float32),
        pltpu.VMEM((_RPT, _H), jnp.float32),
        pltpu.VMEM_SHARED((_NP, _H), jnp.float32),
        [pltpu.SemaphoreType.DMA] * _NBUF,
    ],
)
def _edge_segsum(y_hbm, src_hbm, dst_hbm, out_hbm, src_v, dst_v, rows_v, buf_v,
                 acc_sh, sems):
    cid = lax.axis_index("c")
    sid = lax.axis_index("s")
    wid = sid * _NC + cid

    pltpu.sync_copy(src_hbm.at[wid], src_v)
    pltpu.sync_copy(dst_hbm.at[wid], dst_v)

    def _zero_row(i, carry):
        buf_v[i] = jnp.zeros((_H,), jnp.float32)
        return carry

    lax.fori_loop(0, _RPT, _zero_row, 0)
    pltpu.sync_copy(buf_v, acc_sh.at[pl.ds(sid * _RPT, _RPT)])
    plsc.subcore_barrier()

    for b in range(_NBUF):
        pltpu.async_copy(y_hbm.at[src_v.at[b]], rows_v.at[b], sems[b])

    @pl.loop(0, _NCHUNK, step=_NBUF)
    def _chunks(c0):
        for b in range(_NBUF):
            c = c0 + b
            pltpu.make_async_copy(y_hbm.at[src_v.at[c]], rows_v.at[b],
                                  sems[b]).wait()
            pltpu.sync_copy(rows_v.at[b], acc_sh.at[dst_v.at[c]], add=True)
            nc = c + _NBUF

            @pl.when(nc < _NCHUNK)
            def _prefetch():
                pltpu.async_copy(y_hbm.at[src_v.at[nc]], rows_v.at[b], sems[b])

    plsc.subcore_barrier()

    r0 = sid * _RPT
    pltpu.sync_copy(acc_sh.at[pl.ds(r0, _RPT)], buf_v)
    pltpu.sync_copy(buf_v, out_hbm.at[pl.ds(cid * _NP + r0, _RPT)])


def _mm1_body(x_ref, w_ref, o_ref):
    o_ref[...] = jnp.dot(x_ref[...], w_ref[...], preferred_element_type=jnp.float32)


_mm1 = pl.pallas_call(
    _mm1_body,
    out_shape=jax.ShapeDtypeStruct((_N, _H), jnp.float32),
)


def _mid_body(y_ref, s_ref, b1_ref, w2_ref, z_ref):
    h = y_ref[...] + s_ref[: _N, :] + s_ref[_NP : _NP + _N, :] + b1_ref[...]
    h = jnp.maximum(h, 0.0)
    z_ref[...] = jnp.dot(h, w2_ref[...], preferred_element_type=jnp.float32)


_mid = pl.pallas_call(
    _mid_body,
    out_shape=jax.ShapeDtypeStruct((_N, _H), jnp.float32),
)


def _final_body(z_ref, s_ref, b2_ref, batch_ref, wfc_ref, bfc_ref, o_ref):
    h2 = z_ref[...] + s_ref[: _N, :] + s_ref[_NP : _NP + _N, :] + b2_ref[...]
    h2 = jnp.maximum(h2, 0.0)
    gid = lax.broadcasted_iota(jnp.int32, (_G, _N), 0)
    mask = jnp.where(batch_ref[...] == gid, 1.0, 0.0)
    sums = jnp.dot(mask, h2, preferred_element_type=jnp.float32)
    counts = jnp.sum(mask, axis=1, keepdims=True)
    pooled = sums / jnp.maximum(counts, 1.0)
    o_ref[...] = (
        jnp.dot(pooled, wfc_ref[...], preferred_element_type=jnp.float32)
        + bfc_ref[...]
    )


_final = pl.pallas_call(
    _final_body,
    out_shape=jax.ShapeDtypeStruct((_G, _C), jnp.float32),
)


def kernel(x, edge_index, batch, W1, b1, W2, b2, Wfc, bfc):
    src = edge_index[0].reshape(_NW, _NCHUNK, _CH)
    dst = edge_index[1].reshape(_NW, _NCHUNK, _CH)
    y = _mm1(x, W1)
    s1 = _edge_segsum(y, src, dst)
    z = _mid(y, s1, b1.reshape(1, _H), W2)
    s2 = _edge_segsum(z, src, dst)
    return _final(z, s2, b2.reshape(1, _H), batch.reshape(1, _N), Wfc,
                  bfc.reshape(1, _C))


# speedup vs baseline: 22.3606x; 1.0013x over previous
import functools

import jax
import jax.numpy as jnp
from jax import lax
from jax.experimental import pallas as pl
from jax.experimental.pallas import tpu as pltpu
from jax.experimental.pallas import tpu_sc as plsc

_N = 10000
_E = 320000
_F = 128
_H = 16
_C = 10
_G = 64

_NC = 2
_NS = 16
_NW = _NC * _NS
_EPW = _E // _NW
_CH = 80
_NCHUNK = _EPW // _CH
_NBUF = 5
_NP = 10240
_RPT = _NP // _NS

_sc_mesh = plsc.VectorSubcoreMesh(core_axis_name="c", subcore_axis_name="s")


@functools.partial(
    pl.kernel,
    mesh=_sc_mesh,
    compiler_params=pltpu.CompilerParams(use_tc_tiling_on_sc=False),
    out_type=jax.ShapeDtypeStruct((_NC * _NP, _H), jnp.float32),
    scratch_types=[
        pltpu.VMEM((_EPW,), jnp.int32),
        pltpu.VMEM((_EPW,), jnp.int32),
        pltpu.VMEM((_NBUF, _CH, _H), jnp.float32),
        pltpu.VMEM((_RPT, _H), jnp.float32),
        pltpu.VMEM_SHARED((_NP, _H), jnp.float32),
        [pltpu.SemaphoreType.DMA] * _NBUF,
    ],
)
def _edge_segsum(y_hbm, src_hbm, dst_hbm, out_hbm, src_v, dst_v, rows_v, buf_v,
                 acc_sh, sems):
    cid = lax.axis_index("c")
    sid = lax.axis_index("s")
    wid = sid * _NC + cid

    pltpu.sync_copy(src_hbm.at[pl.ds(wid * _EPW, _EPW)], src_v)
    pltpu.sync_copy(dst_hbm.at[pl.ds(wid * _EPW, _EPW)], dst_v)

    def _zero_row(i, carry):
        buf_v[i] = jnp.zeros((_H,), jnp.float32)
        return carry

    lax.fori_loop(0, _RPT, _zero_row, 0)
    pltpu.sync_copy(buf_v, acc_sh.at[pl.ds(sid * _RPT, _RPT)])
    plsc.subcore_barrier()

    for b in range(_NBUF):
        pltpu.async_copy(y_hbm.at[src_v.at[pl.ds(b * _CH, _CH)]],
                         rows_v.at[b], sems[b])

    @pl.loop(0, _NCHUNK, step=_NBUF)
    def _chunks(c0):
        for b in range(_NBUF):
            c = c0 + b
            pltpu.make_async_copy(
                y_hbm.at[src_v.at[pl.ds(c * _CH, _CH)]], rows_v.at[b],
                sems[b]).wait()
            pltpu.sync_copy(rows_v.at[b],
                            acc_sh.at[dst_v.at[pl.ds(c * _CH, _CH)]],
                            add=True)
            nc = c + _NBUF

            @pl.when(nc < _NCHUNK)
            def _prefetch():
                pltpu.async_copy(y_hbm.at[src_v.at[pl.ds(nc * _CH, _CH)]],
                                 rows_v.at[b], sems[b])

    plsc.subcore_barrier()

    r0 = sid * _RPT
    pltpu.sync_copy(acc_sh.at[pl.ds(r0, _RPT)], buf_v)
    pltpu.sync_copy(buf_v, out_hbm.at[pl.ds(cid * _NP + r0, _RPT)])


def _mm1_body(x_ref, w_ref, o_ref):
    o_ref[...] = jnp.dot(x_ref[...], w_ref[...], preferred_element_type=jnp.float32)


_mm1 = pl.pallas_call(
    _mm1_body,
    out_shape=jax.ShapeDtypeStruct((_N, _H), jnp.float32),
)


def _mid_body(y_ref, s_ref, b1_ref, w2_ref, z_ref):
    h = y_ref[...] + s_ref[: _N, :] + s_ref[_NP : _NP + _N, :] + b1_ref[...]
    h = jnp.maximum(h, 0.0)
    z_ref[...] = jnp.dot(h, w2_ref[...], preferred_element_type=jnp.float32)


_mid = pl.pallas_call(
    _mid_body,
    out_shape=jax.ShapeDtypeStruct((_N, _H), jnp.float32),
)


def _final_body(z_ref, s_ref, b2_ref, batch_ref, wfc_ref, bfc_ref, o_ref):
    h2 = z_ref[...] + s_ref[: _N, :] + s_ref[_NP : _NP + _N, :] + b2_ref[...]
    h2 = jnp.maximum(h2, 0.0)
    gid = lax.broadcasted_iota(jnp.int32, (_G, _N), 0)
    mask = jnp.where(batch_ref[...] == gid, 1.0, 0.0)
    sums = jnp.dot(mask, h2, preferred_element_type=jnp.float32)
    counts = jnp.sum(mask, axis=1, keepdims=True)
    pooled = sums / jnp.maximum(counts, 1.0)
    o_ref[...] = (
        jnp.dot(pooled, wfc_ref[...], preferred_element_type=jnp.float32)
        + bfc_ref[...]
    )


_final = pl.pallas_call(
    _final_body,
    out_shape=jax.ShapeDtypeStruct((_G, _C), jnp.float32),
)


def kernel(x, edge_index, batch, W1, b1, W2, b2, Wfc, bfc):
    src = edge_index[0]
    dst = edge_index[1]
    y = _mm1(x, W1)
    s1 = _edge_segsum(y, src, dst)
    z = _mid(y, s1, b1.reshape(1, _H), W2)
    s2 = _edge_segsum(z, src, dst)
    return _final(z, s2, b2.reshape(1, _H), batch.reshape(1, _N), Wfc,
                  bfc.reshape(1, _C))


# speedup vs baseline: 22.8523x; 1.0220x over previous
import functools

import jax
import jax.numpy as jnp
from jax import lax
from jax.experimental import pallas as pl
from jax.experimental.pallas import tpu as pltpu
from jax.experimental.pallas import tpu_sc as plsc

_N = 10000
_E = 320000
_F = 128
_H = 16
_C = 10
_G = 64

_NC = 2
_NS = 16
_NW = _NC * _NS
_EPW = _E // _NW
_CH = 80
_NCHUNK = _EPW // _CH
_NBUF = 4
_NSLOT = 2 * _NBUF
_NP = 10240
_RPT = _NP // _NS

_sc_mesh = plsc.VectorSubcoreMesh(core_axis_name="c", subcore_axis_name="s")


@functools.partial(
    pl.kernel,
    mesh=_sc_mesh,
    compiler_params=pltpu.CompilerParams(use_tc_tiling_on_sc=False),
    out_type=jax.ShapeDtypeStruct((_NC * _NP, _H), jnp.float32),
    scratch_types=[
        pltpu.VMEM((_EPW,), jnp.int32),
        pltpu.VMEM((_EPW,), jnp.int32),
        pltpu.VMEM((_NSLOT, _CH, _H), jnp.float32),
        pltpu.VMEM((_RPT, _H), jnp.float32),
        pltpu.VMEM_SHARED((_NP, _H), jnp.float32),
        [pltpu.SemaphoreType.DMA] * _NSLOT,
        [pltpu.SemaphoreType.DMA] * _NSLOT,
    ],
)
def _edge_segsum(y_hbm, src_hbm, dst_hbm, out_hbm, src_v, dst_v, rows_v, buf_v,
                 acc_sh, gsems, ssems):
    cid = lax.axis_index("c")
    sid = lax.axis_index("s")
    wid = sid * _NC + cid

    pltpu.sync_copy(src_hbm.at[pl.ds(wid * _EPW, _EPW)], src_v)
    pltpu.sync_copy(dst_hbm.at[pl.ds(wid * _EPW, _EPW)], dst_v)

    def _zero_row(i, carry):
        buf_v[i] = jnp.zeros((_H,), jnp.float32)
        return carry

    lax.fori_loop(0, _RPT, _zero_row, 0)
    pltpu.sync_copy(buf_v, acc_sh.at[pl.ds(sid * _RPT, _RPT)])
    plsc.subcore_barrier()

    for k in range(_NBUF):
        pltpu.async_copy(y_hbm.at[src_v.at[pl.ds(k * _CH, _CH)]],
                         rows_v.at[k], gsems[k])

    @pl.loop(0, _NCHUNK)
    def _chunks(c):
        nc = c + _NBUF

        @pl.when(nc < _NCHUNK)
        def _refill():
            bf = lax.rem(nc, _NSLOT)
            for s in range(_NSLOT):
                @pl.when(bf == s)
                def _do():
                    @pl.when(c >= _NBUF)
                    def _drain():
                        pltpu.make_async_copy(
                            rows_v.at[s], acc_sh.at[dst_v.at[pl.ds(0, _CH)]],
                            ssems[s]).wait()
                    pltpu.async_copy(
                        y_hbm.at[src_v.at[pl.ds(nc * _CH, _CH)]],
                        rows_v.at[s], gsems[s])

        b = lax.rem(c, _NSLOT)
        for s in range(_NSLOT):
            @pl.when(b == s)
            def _consume():
                pltpu.make_async_copy(
                    y_hbm.at[src_v.at[pl.ds(c * _CH, _CH)]], rows_v.at[s],
                    gsems[s]).wait()
                pltpu.async_copy(rows_v.at[s],
                                 acc_sh.at[dst_v.at[pl.ds(c * _CH, _CH)]],
                                 ssems[s], add=True)

    for s in range(_NSLOT):
        pltpu.make_async_copy(rows_v.at[s],
                              acc_sh.at[dst_v.at[pl.ds(0, _CH)]],
                              ssems[s]).wait()

    plsc.subcore_barrier()

    r0 = sid * _RPT
    pltpu.sync_copy(acc_sh.at[pl.ds(r0, _RPT)], buf_v)
    pltpu.sync_copy(buf_v, out_hbm.at[pl.ds(cid * _NP + r0, _RPT)])


def _mm1_body(x_ref, w_ref, o_ref):
    o_ref[...] = jnp.dot(x_ref[...], w_ref[...], preferred_element_type=jnp.float32)


_mm1 = pl.pallas_call(
    _mm1_body,
    out_shape=jax.ShapeDtypeStruct((_N, _H), jnp.float32),
)


def _mid_body(y_ref, s_ref, b1_ref, w2_ref, z_ref):
    h = y_ref[...] + s_ref[: _N, :] + s_ref[_NP : _NP + _N, :] + b1_ref[...]
    h = jnp.maximum(h, 0.0)
    z_ref[...] = jnp.dot(h, w2_ref[...], preferred_element_type=jnp.float32)


_mid = pl.pallas_call(
    _mid_body,
    out_shape=jax.ShapeDtypeStruct((_N, _H), jnp.float32),
)


def _final_body(z_ref, s_ref, b2_ref, batch_ref, wfc_ref, bfc_ref, o_ref):
    h2 = z_ref[...] + s_ref[: _N, :] + s_ref[_NP : _NP + _N, :] + b2_ref[...]
    h2 = jnp.maximum(h2, 0.0)
    gid = lax.broadcasted_iota(jnp.int32, (_G, _N), 0)
    mask = jnp.where(batch_ref[...] == gid, 1.0, 0.0)
    sums = jnp.dot(mask, h2, preferred_element_type=jnp.float32)
    counts = jnp.sum(mask, axis=1, keepdims=True)
    pooled = sums / jnp.maximum(counts, 1.0)
    o_ref[...] = (
        jnp.dot(pooled, wfc_ref[...], preferred_element_type=jnp.float32)
        + bfc_ref[...]
    )


_final = pl.pallas_call(
    _final_body,
    out_shape=jax.ShapeDtypeStruct((_G, _C), jnp.float32),
)


def kernel(x, edge_index, batch, W1, b1, W2, b2, Wfc, bfc):
    src = edge_index[0]
    dst = edge_index[1]
    y = _mm1(x, W1)
    s1 = _edge_segsum(y, src, dst)
    z = _mid(y, s1, b1.reshape(1, _H), W2)
    s2 = _edge_segsum(z, src, dst)
    return _final(z, s2, b2.reshape(1, _H), batch.reshape(1, _N), Wfc,
                  bfc.reshape(1, _C))


# speedup vs baseline: 26.9755x; 1.1804x over previous
import functools

import jax
import jax.numpy as jnp
from jax import lax
from jax.experimental import pallas as pl
from jax.experimental.pallas import tpu as pltpu
from jax.experimental.pallas import tpu_sc as plsc

_N = 10000
_E = 320000
_F = 128
_H = 16
_C = 10
_G = 64

_NC = 2
_NS = 16
_NW = _NC * _NS
_EPW = _E // _NW
_CH = 128
_NCHUNK = _EPW // _CH
_TAIL = _EPW - _NCHUNK * _CH
_NBUF = 4
_NSLOT = 2 * _NBUF
_NP = 10240
_RPT = _NP // _NS

_sc_mesh = plsc.VectorSubcoreMesh(core_axis_name="c", subcore_axis_name="s")


@functools.partial(
    pl.kernel,
    mesh=_sc_mesh,
    compiler_params=pltpu.CompilerParams(use_tc_tiling_on_sc=False),
    out_type=jax.ShapeDtypeStruct((_NC * _NP, _H), jnp.float32),
    scratch_types=[
        pltpu.VMEM((_EPW,), jnp.int32),
        pltpu.VMEM((_EPW,), jnp.int32),
        pltpu.VMEM((_NSLOT, _CH, _H), jnp.float32),
        pltpu.VMEM((_RPT, _H), jnp.float32),
        pltpu.VMEM_SHARED((_NP, _H), jnp.float32),
        [pltpu.SemaphoreType.DMA] * _NSLOT,
        [pltpu.SemaphoreType.DMA] * _NSLOT,
    ],
)
def _edge_segsum(y_hbm, ei_hbm, out_hbm, src_v, dst_v, rows_v, buf_v,
                 acc_sh, gsems, ssems):
    cid = lax.axis_index("c")
    sid = lax.axis_index("s")
    wid = sid * _NC + cid

    pltpu.sync_copy(ei_hbm.at[0, pl.ds(wid * _EPW, _EPW)], src_v)
    pltpu.sync_copy(ei_hbm.at[1, pl.ds(wid * _EPW, _EPW)], dst_v)

    def _zero_row(i, carry):
        buf_v[i] = jnp.zeros((_H,), jnp.float32)
        return carry

    lax.fori_loop(0, _RPT, _zero_row, 0)
    pltpu.sync_copy(buf_v, acc_sh.at[pl.ds(sid * _RPT, _RPT)])
    plsc.subcore_barrier()

    for k in range(_NBUF):
        pltpu.async_copy(y_hbm.at[src_v.at[pl.ds(k * _CH, _CH)]],
                         rows_v.at[k], gsems[k])

    @pl.loop(0, _NCHUNK)
    def _chunks(c):
        nc = c + _NBUF

        @pl.when(nc < _NCHUNK)
        def _refill():
            bf = lax.rem(nc, _NSLOT)
            for s in range(_NSLOT):
                @pl.when(bf == s)
                def _do():
                    @pl.when(c >= _NBUF)
                    def _drain():
                        pltpu.make_async_copy(
                            rows_v.at[s], acc_sh.at[dst_v.at[pl.ds(0, _CH)]],
                            ssems[s]).wait()
                    pltpu.async_copy(
                        y_hbm.at[src_v.at[pl.ds(nc * _CH, _CH)]],
                        rows_v.at[s], gsems[s])

        b = lax.rem(c, _NSLOT)
        for s in range(_NSLOT):
            @pl.when(b == s)
            def _consume():
                pltpu.make_async_copy(
                    y_hbm.at[src_v.at[pl.ds(c * _CH, _CH)]], rows_v.at[s],
                    gsems[s]).wait()
                pltpu.async_copy(rows_v.at[s],
                                 acc_sh.at[dst_v.at[pl.ds(c * _CH, _CH)]],
                                 ssems[s], add=True)

    for s in range(_NSLOT):
        pltpu.make_async_copy(rows_v.at[s],
                              acc_sh.at[dst_v.at[pl.ds(0, _CH)]],
                              ssems[s]).wait()

    t0 = _NCHUNK * _CH
    pltpu.async_copy(y_hbm.at[src_v.at[pl.ds(t0, _TAIL)]],
                     rows_v.at[0, pl.ds(0, _TAIL)], gsems[0]).wait()
    pltpu.sync_copy(rows_v.at[0, pl.ds(0, _TAIL)],
                    acc_sh.at[dst_v.at[pl.ds(t0, _TAIL)]], add=True)

    plsc.subcore_barrier()

    r0 = sid * _RPT
    pltpu.sync_copy(acc_sh.at[pl.ds(r0, _RPT)], buf_v)
    pltpu.sync_copy(buf_v, out_hbm.at[pl.ds(cid * _NP + r0, _RPT)])


def _mm1_body(x_ref, w_ref, o_ref):
    o_ref[...] = jnp.dot(x_ref[...], w_ref[...], preferred_element_type=jnp.float32)


_mm1 = pl.pallas_call(
    _mm1_body,
    out_shape=jax.ShapeDtypeStruct((_N, _H), jnp.float32),
)


def _mid_body(y_ref, s_ref, b1_ref, w2_ref, z_ref):
    h = y_ref[...] + s_ref[: _N, :] + s_ref[_NP : _NP + _N, :] + b1_ref[...]
    h = jnp.maximum(h, 0.0)
    z_ref[...] = jnp.dot(h, w2_ref[...], preferred_element_type=jnp.float32)


_mid = pl.pallas_call(
    _mid_body,
    out_shape=jax.ShapeDtypeStruct((_N, _H), jnp.float32),
)


def _final_body(z_ref, s_ref, b2_ref, batch_ref, wfc_ref, bfc_ref, o_ref):
    h2 = z_ref[...] + s_ref[: _N, :] + s_ref[_NP : _NP + _N, :] + b2_ref[...]
    h2 = jnp.maximum(h2, 0.0)
    gid = lax.broadcasted_iota(jnp.int32, (_G, _N), 0)
    mask = jnp.where(batch_ref[...] == gid, 1.0, 0.0)
    sums = jnp.dot(mask, h2, preferred_element_type=jnp.float32)
    counts = jnp.sum(mask, axis=1, keepdims=True)
    pooled = sums / jnp.maximum(counts, 1.0)
    o_ref[...] = (
        jnp.dot(pooled, wfc_ref[...], preferred_element_type=jnp.float32)
        + bfc_ref[...]
    )


_final = pl.pallas_call(
    _final_body,
    out_shape=jax.ShapeDtypeStruct((_G, _C), jnp.float32),
)


def kernel(x, edge_index, batch, W1, b1, W2, b2, Wfc, bfc):
    y = _mm1(x, W1)
    s1 = _edge_segsum(y, edge_index)
    z = _mid(y, s1, b1.reshape(1, _H), W2)
    s2 = _edge_segsum(z, edge_index)
    return _final(z, s2, b2.reshape(1, _H), batch.reshape(1, _N), Wfc,
                  bfc.reshape(1, _C))


# speedup vs baseline: 37.3051x; 1.3829x over previous
import functools

import jax
import jax.numpy as jnp
from jax import lax
from jax.experimental import pallas as pl
from jax.experimental.pallas import tpu as pltpu
from jax.experimental.pallas import tpu_sc as plsc

_N = 10000
_E = 320000
_F = 128
_H = 16
_C = 10
_G = 64

_NC = 2
_NS = 16
_NW = _NC * _NS
_EPW = _E // _NW
_CH = 128
_NCHUNK = _EPW // _CH
_TAIL = _EPW - _NCHUNK * _CH
_NBUF = 4
_NSLOT = 2 * _NBUF
_NP = 10240
_RPT = _NP // _NS

_sc_mesh = plsc.VectorSubcoreMesh(core_axis_name="c", subcore_axis_name="s")


@functools.partial(
    pl.kernel,
    mesh=_sc_mesh,
    compiler_params=pltpu.CompilerParams(use_tc_tiling_on_sc=False),
    out_type=jax.ShapeDtypeStruct((_NC * _NP, _H), jnp.float32),
    scratch_types=[
        pltpu.VMEM((_EPW,), jnp.int32),
        pltpu.VMEM((_EPW,), jnp.int32),
        pltpu.VMEM((_NSLOT, _CH, _H), jnp.float32),
        pltpu.VMEM((_RPT, _H), jnp.float32),
        pltpu.VMEM_SHARED((_NP, _H), jnp.float32),
        [pltpu.SemaphoreType.DMA] * _NSLOT,
        [pltpu.SemaphoreType.DMA] * _NSLOT,
    ],
)
def _edge_segsum(y_hbm, ei_hbm, out_hbm, src_v, dst_v, rows_v, buf_v,
                 acc_sh, gsems, ssems):
    cid = lax.axis_index("c")
    sid = lax.axis_index("s")
    wid = sid * _NC + cid

    pltpu.sync_copy(ei_hbm.at[0, pl.ds(wid * _EPW, _EPW)], src_v)
    pltpu.sync_copy(ei_hbm.at[1, pl.ds(wid * _EPW, _EPW)], dst_v)

    def _zero_row(i, carry):
        buf_v[i] = jnp.zeros((_H,), jnp.float32)
        return carry

    lax.fori_loop(0, _RPT, _zero_row, 0)
    pltpu.sync_copy(buf_v, acc_sh.at[pl.ds(sid * _RPT, _RPT)])
    plsc.subcore_barrier()

    for k in range(_NBUF):
        pltpu.async_copy(y_hbm.at[src_v.at[pl.ds(k * _CH, _CH)]],
                         rows_v.at[k], gsems[k])

    @pl.loop(0, _NCHUNK)
    def _chunks(c):
        nc = c + _NBUF

        @pl.when(nc < _NCHUNK)
        def _refill():
            bf = lax.rem(nc, _NSLOT)
            for s in range(_NSLOT):
                @pl.when(bf == s)
                def _do():
                    @pl.when(c >= _NBUF)
                    def _drain():
                        pltpu.make_async_copy(
                            rows_v.at[s], acc_sh.at[dst_v.at[pl.ds(0, _CH)]],
                            ssems[s]).wait()
                    pltpu.async_copy(
                        y_hbm.at[src_v.at[pl.ds(nc * _CH, _CH)]],
                        rows_v.at[s], gsems[s])

        b = lax.rem(c, _NSLOT)
        for s in range(_NSLOT):
            @pl.when(b == s)
            def _consume():
                pltpu.make_async_copy(
                    y_hbm.at[src_v.at[pl.ds(c * _CH, _CH)]], rows_v.at[s],
                    gsems[s]).wait()
                pltpu.async_copy(rows_v.at[s],
                                 acc_sh.at[dst_v.at[pl.ds(c * _CH, _CH)]],
                                 ssems[s], add=True)

    for s in range(_NSLOT):
        pltpu.make_async_copy(rows_v.at[s],
                              acc_sh.at[dst_v.at[pl.ds(0, _CH)]],
                              ssems[s]).wait()

    t0 = _NCHUNK * _CH
    pltpu.async_copy(y_hbm.at[src_v.at[pl.ds(t0, _TAIL)]],
                     rows_v.at[0, pl.ds(0, _TAIL)], gsems[0]).wait()
    pltpu.sync_copy(rows_v.at[0, pl.ds(0, _TAIL)],
                    acc_sh.at[dst_v.at[pl.ds(t0, _TAIL)]], add=True)

    plsc.subcore_barrier()

    r0 = sid * _RPT
    pltpu.sync_copy(acc_sh.at[pl.ds(r0, _RPT)], buf_v)
    pltpu.sync_copy(buf_v, out_hbm.at[pl.ds(cid * _NP + r0, _RPT)])


_NR = _N // 8
_NPR = _NP // 8


def _mm1_body(x_ref, w_ref, o_ref):
    y = jnp.dot(x_ref[...], w_ref[...], preferred_element_type=jnp.float32)
    y3 = y.reshape(_NR, 8, _H)
    o_ref[...] = jnp.concatenate([y3[:, a, :] for a in range(8)], axis=1)


_mm1 = pl.pallas_call(
    _mm1_body,
    out_shape=jax.ShapeDtypeStruct((_NR, 128), jnp.float32),
)


def _mid_body(yp_ref, sp_ref, b1p_ref, w2blk_ref, zp_ref):
    h = (yp_ref[...] + sp_ref[:_NR, :] + sp_ref[_NPR : _NPR + _NR, :]
         + b1p_ref[...])
    h = jnp.maximum(h, 0.0)
    zp_ref[...] = jnp.dot(h, w2blk_ref[...], preferred_element_type=jnp.float32)


_mid = pl.pallas_call(
    _mid_body,
    out_shape=jax.ShapeDtypeStruct((_NR, 128), jnp.float32),
)


def _final_body(zp_ref, sp_ref, b2p_ref, batcht_ref, wfc_ref, bfc_ref, o_ref):
    h2 = (zp_ref[...] + sp_ref[:_NR, :] + sp_ref[_NPR : _NPR + _NR, :]
          + b2p_ref[...])
    h2 = jnp.maximum(h2, 0.0)
    gid = lax.broadcasted_iota(jnp.int32, (_G, _NR), 0)
    sums = jnp.zeros((_G, _H), jnp.float32)
    counts = jnp.zeros((_G, 1), jnp.float32)
    for a in range(8):
        mask = jnp.where(batcht_ref[a : a + 1, :] == gid, 1.0, 0.0)
        sums = sums + jnp.dot(mask, h2[:, 16 * a : 16 * (a + 1)],
                              preferred_element_type=jnp.float32)
        counts = counts + jnp.sum(mask, axis=1, keepdims=True)
    pooled = sums / jnp.maximum(counts, 1.0)
    o_ref[...] = (
        jnp.dot(pooled, wfc_ref[...], preferred_element_type=jnp.float32)
        + bfc_ref[...]
    )


_final = pl.pallas_call(
    _final_body,
    out_shape=jax.ShapeDtypeStruct((_G, _C), jnp.float32),
)


def kernel(x, edge_index, batch, W1, b1, W2, b2, Wfc, bfc):
    w2blk = jnp.kron(jnp.eye(8, dtype=jnp.float32), W2)
    b1p = jnp.tile(b1, 8).reshape(1, 128)
    b2p = jnp.tile(b2, 8).reshape(1, 128)
    batcht = batch.reshape(_NR, 8).T
    yp = _mm1(x, W1)
    s1 = _edge_segsum(yp.reshape(_N, _H), edge_index)
    zp = _mid(yp, s1.reshape(2 * _NPR, 128), b1p, w2blk)
    s2 = _edge_segsum(zp.reshape(_N, _H), edge_index)
    return _final(zp, s2.reshape(2 * _NPR, 128), b2p, batcht, Wfc,
                  bfc.reshape(1, _C))


# speedup vs baseline: 39.0552x; 1.0469x over previous
import functools

import jax
import jax.numpy as jnp
from jax import lax
from jax.experimental import pallas as pl
from jax.experimental.pallas import tpu as pltpu
from jax.experimental.pallas import tpu_sc as plsc

_N = 10000
_E = 320000
_F = 128
_H = 16
_C = 10
_G = 64

_NC = 2
_NS = 16
_NW = _NC * _NS
_EPW = _E // _NW
_CH = 128
_NCHUNK = _EPW // _CH
_TAIL = _EPW - _NCHUNK * _CH
_NBUF = 4
_NSLOT = 2 * _NBUF
_NP = 10240
_RPT = _NP // _NS

_sc_mesh = plsc.VectorSubcoreMesh(core_axis_name="c", subcore_axis_name="s")


@functools.partial(
    pl.kernel,
    mesh=_sc_mesh,
    compiler_params=pltpu.CompilerParams(use_tc_tiling_on_sc=False),
    out_type=jax.ShapeDtypeStruct((_NC * _NP, _H), jnp.float32),
    scratch_types=[
        pltpu.VMEM((_EPW,), jnp.int32),
        pltpu.VMEM((_EPW,), jnp.int32),
        pltpu.VMEM((_NSLOT, _CH, _H), jnp.float32),
        pltpu.VMEM((_RPT, _H), jnp.float32),
        pltpu.VMEM_SHARED((_NP, _H), jnp.float32),
        [pltpu.SemaphoreType.DMA] * _NSLOT,
        [pltpu.SemaphoreType.DMA] * _NSLOT,
    ],
)
def _edge_segsum(y_hbm, ei_hbm, out_hbm, src_v, dst_v, rows_v, buf_v,
                 acc_sh, gsems, ssems):
    cid = lax.axis_index("c")
    sid = lax.axis_index("s")
    wid = sid * _NC + cid

    idx_cp0 = pltpu.async_copy(ei_hbm.at[0, pl.ds(wid * _EPW, _EPW)], src_v,
                               gsems[0])
    idx_cp1 = pltpu.async_copy(ei_hbm.at[1, pl.ds(wid * _EPW, _EPW)], dst_v,
                               gsems[1])

    def _zero_row(i, carry):
        buf_v[i] = jnp.zeros((_H,), jnp.float32)
        return carry

    lax.fori_loop(0, _RPT, _zero_row, 0)
    pltpu.sync_copy(buf_v, acc_sh.at[pl.ds(sid * _RPT, _RPT)])
    idx_cp0.wait()
    idx_cp1.wait()
    plsc.subcore_barrier()

    for k in range(_NBUF):
        pltpu.async_copy(y_hbm.at[src_v.at[pl.ds(k * _CH, _CH)]],
                         rows_v.at[k], gsems[k])

    @pl.loop(0, _NCHUNK)
    def _chunks(c):
        nc = c + _NBUF

        @pl.when(nc < _NCHUNK)
        def _refill():
            bf = lax.rem(nc, _NSLOT)
            for s in range(_NSLOT):
                @pl.when(bf == s)
                def _do():
                    @pl.when(c >= _NBUF)
                    def _drain():
                        pltpu.make_async_copy(
                            rows_v.at[s], acc_sh.at[dst_v.at[pl.ds(0, _CH)]],
                            ssems[s]).wait()
                    pltpu.async_copy(
                        y_hbm.at[src_v.at[pl.ds(nc * _CH, _CH)]],
                        rows_v.at[s], gsems[s])

        b = lax.rem(c, _NSLOT)
        for s in range(_NSLOT):
            @pl.when(b == s)
            def _consume():
                pltpu.make_async_copy(
                    y_hbm.at[src_v.at[pl.ds(c * _CH, _CH)]], rows_v.at[s],
                    gsems[s]).wait()
                pltpu.async_copy(rows_v.at[s],
                                 acc_sh.at[dst_v.at[pl.ds(c * _CH, _CH)]],
                                 ssems[s], add=True)

    for s in range(_NSLOT):
        pltpu.make_async_copy(rows_v.at[s],
                              acc_sh.at[dst_v.at[pl.ds(0, _CH)]],
                              ssems[s]).wait()

    t0 = _NCHUNK * _CH
    pltpu.async_copy(y_hbm.at[src_v.at[pl.ds(t0, _TAIL)]],
                     rows_v.at[0, pl.ds(0, _TAIL)], gsems[0]).wait()
    pltpu.sync_copy(rows_v.at[0, pl.ds(0, _TAIL)],
                    acc_sh.at[dst_v.at[pl.ds(t0, _TAIL)]], add=True)

    plsc.subcore_barrier()

    r0 = sid * _RPT
    pltpu.sync_copy(acc_sh.at[pl.ds(r0, _RPT)], buf_v)
    pltpu.sync_copy(buf_v, out_hbm.at[pl.ds(cid * _NP + r0, _RPT)])


_NR = _N // 8
_NPR = _NP // 8


_MMB = 2048


def _mm1_body(x_ref, w_ref, o_ref):
    y = jnp.dot(x_ref[...], w_ref[...], preferred_element_type=jnp.float32)
    y3 = y.reshape(_MMB // 8, 8, _H)
    o_ref[...] = jnp.concatenate([y3[:, a, :] for a in range(8)], axis=1)


_mm1 = pl.pallas_call(
    _mm1_body,
    grid=((_N + _MMB - 1) // _MMB,),
    in_specs=[
        pl.BlockSpec((_MMB, _F), lambda i: (i, 0)),
        pl.BlockSpec((_F, _H), lambda i: (0, 0)),
    ],
    out_specs=pl.BlockSpec((_MMB // 8, 128), lambda i: (i, 0)),
    out_shape=jax.ShapeDtypeStruct((_NR, 128), jnp.float32),
)


def _mid_body(yp_ref, sp_ref, b1p_ref, w2blk_ref, zp_ref):
    h = (yp_ref[...] + sp_ref[:_NR, :] + sp_ref[_NPR : _NPR + _NR, :]
         + b1p_ref[...])
    h = jnp.maximum(h, 0.0)
    zp_ref[...] = jnp.dot(h, w2blk_ref[...], preferred_element_type=jnp.float32)


_mid = pl.pallas_call(
    _mid_body,
    out_shape=jax.ShapeDtypeStruct((_NR, 128), jnp.float32),
)


def _final_body(zp_ref, sp_ref, b2p_ref, batcht_ref, wfc_ref, bfc_ref, o_ref):
    h2 = (zp_ref[...] + sp_ref[:_NR, :] + sp_ref[_NPR : _NPR + _NR, :]
          + b2p_ref[...])
    h2 = jnp.maximum(h2, 0.0)
    gid = lax.broadcasted_iota(jnp.int32, (_G, _NR), 0)
    sums = jnp.zeros((_G, _H), jnp.float32)
    counts = jnp.zeros((_G, 1), jnp.float32)
    for a in range(8):
        mask = jnp.where(batcht_ref[a : a + 1, :] == gid, 1.0, 0.0)
        sums = sums + jnp.dot(mask, h2[:, 16 * a : 16 * (a + 1)],
                              preferred_element_type=jnp.float32)
        counts = counts + jnp.sum(mask, axis=1, keepdims=True)
    pooled = sums / jnp.maximum(counts, 1.0)
    o_ref[...] = (
        jnp.dot(pooled, wfc_ref[...], preferred_element_type=jnp.float32)
        + bfc_ref[...]
    )


_final = pl.pallas_call(
    _final_body,
    out_shape=jax.ShapeDtypeStruct((_G, _C), jnp.float32),
)


def kernel(x, edge_index, batch, W1, b1, W2, b2, Wfc, bfc):
    w2blk = jnp.kron(jnp.eye(8, dtype=jnp.float32), W2)
    b1p = jnp.tile(b1, 8).reshape(1, 128)
    b2p = jnp.tile(b2, 8).reshape(1, 128)
    batcht = batch.reshape(_NR, 8).T
    yp = _mm1(x, W1)
    s1 = _edge_segsum(yp.reshape(_N, _H), edge_index)
    zp = _mid(yp, s1.reshape(2 * _NPR, 128), b1p, w2blk)
    s2 = _edge_segsum(zp.reshape(_N, _H), edge_index)
    return _final(zp, s2.reshape(2 * _NPR, 128), b2p, batcht, Wfc,
                  bfc.reshape(1, _C))


# speedup vs baseline: 41.6656x; 1.0668x over previous
import functools

import jax
import jax.numpy as jnp
from jax import lax
from jax.experimental import pallas as pl
from jax.experimental.pallas import tpu as pltpu
from jax.experimental.pallas import tpu_sc as plsc

_N = 10000
_E = 320000
_F = 128
_H = 16
_C = 10
_G = 64

_NC = 2
_NS = 16
_NW = _NC * _NS
_EPW = _E // _NW
_CH = 128
_NCHUNK = _EPW // _CH
_TAIL = _EPW - _NCHUNK * _CH
_NBUF = 6
_NSLOT = 2 * _NBUF
_NP = 10240
_RPT = _NP // _NS

_sc_mesh = plsc.VectorSubcoreMesh(core_axis_name="c", subcore_axis_name="s")


@functools.partial(
    pl.kernel,
    mesh=_sc_mesh,
    compiler_params=pltpu.CompilerParams(use_tc_tiling_on_sc=False),
    out_type=jax.ShapeDtypeStruct((_NC * _NP, _H), jnp.float32),
    scratch_types=[
        pltpu.VMEM((_EPW,), jnp.int32),
        pltpu.VMEM((_EPW,), jnp.int32),
        pltpu.VMEM((_NSLOT, _CH, _H), jnp.float32),
        pltpu.VMEM((_RPT, _H), jnp.float32),
        pltpu.VMEM_SHARED((_NP, _H), jnp.float32),
        [pltpu.SemaphoreType.DMA] * _NSLOT,
        [pltpu.SemaphoreType.DMA] * _NSLOT,
    ],
)
def _edge_segsum(y_hbm, ei_hbm, out_hbm, src_v, dst_v, rows_v, buf_v,
                 acc_sh, gsems, ssems):
    cid = lax.axis_index("c")
    sid = lax.axis_index("s")
    wid = sid * _NC + cid

    idx_cp0 = pltpu.async_copy(ei_hbm.at[0, pl.ds(wid * _EPW, _EPW)], src_v,
                               gsems[0])
    idx_cp1 = pltpu.async_copy(ei_hbm.at[1, pl.ds(wid * _EPW, _EPW)], dst_v,
                               gsems[1])

    def _zero_row(i, carry):
        buf_v[i] = jnp.zeros((_H,), jnp.float32)
        return carry

    lax.fori_loop(0, _RPT, _zero_row, 0)
    pltpu.sync_copy(buf_v, acc_sh.at[pl.ds(sid * _RPT, _RPT)])
    idx_cp0.wait()
    idx_cp1.wait()
    plsc.subcore_barrier()

    for k in range(_NBUF):
        pltpu.async_copy(y_hbm.at[src_v.at[pl.ds(k * _CH, _CH)]],
                         rows_v.at[k], gsems[k])

    @pl.loop(0, _NCHUNK)
    def _chunks(c):
        nc = c + _NBUF

        @pl.when(nc < _NCHUNK)
        def _refill():
            bf = lax.rem(nc, _NSLOT)
            for s in range(_NSLOT):
                @pl.when(bf == s)
                def _do():
                    @pl.when(c >= _NBUF)
                    def _drain():
                        pltpu.make_async_copy(
                            rows_v.at[s], acc_sh.at[dst_v.at[pl.ds(0, _CH)]],
                            ssems[s]).wait()
                    pltpu.async_copy(
                        y_hbm.at[src_v.at[pl.ds(nc * _CH, _CH)]],
                        rows_v.at[s], gsems[s])

        b = lax.rem(c, _NSLOT)
        for s in range(_NSLOT):
            @pl.when(b == s)
            def _consume():
                pltpu.make_async_copy(
                    y_hbm.at[src_v.at[pl.ds(c * _CH, _CH)]], rows_v.at[s],
                    gsems[s]).wait()
                pltpu.async_copy(rows_v.at[s],
                                 acc_sh.at[dst_v.at[pl.ds(c * _CH, _CH)]],
                                 ssems[s], add=True)

    for s in range(_NSLOT):
        pltpu.make_async_copy(rows_v.at[s],
                              acc_sh.at[dst_v.at[pl.ds(0, _CH)]],
                              ssems[s]).wait()

    t0 = _NCHUNK * _CH
    pltpu.async_copy(y_hbm.at[src_v.at[pl.ds(t0, _TAIL)]],
                     rows_v.at[0, pl.ds(0, _TAIL)], gsems[0]).wait()
    pltpu.sync_copy(rows_v.at[0, pl.ds(0, _TAIL)],
                    acc_sh.at[dst_v.at[pl.ds(t0, _TAIL)]], add=True)

    plsc.subcore_barrier()

    r0 = sid * _RPT
    pltpu.sync_copy(acc_sh.at[pl.ds(r0, _RPT)], buf_v)
    pltpu.sync_copy(buf_v, out_hbm.at[pl.ds(cid * _NP + r0, _RPT)])


_NR = _N // 8
_NPR = _NP // 8


_MMB = 2048


def _mm1_body(x_ref, w_ref, o_ref):
    y = jnp.dot(x_ref[...], w_ref[...], preferred_element_type=jnp.float32)
    y3 = y.reshape(_MMB // 8, 8, _H)
    o_ref[...] = jnp.concatenate([y3[:, a, :] for a in range(8)], axis=1)


_mm1 = pl.pallas_call(
    _mm1_body,
    grid=((_N + _MMB - 1) // _MMB,),
    in_specs=[
        pl.BlockSpec((_MMB, _F), lambda i: (i, 0)),
        pl.BlockSpec((_F, _H), lambda i: (0, 0)),
    ],
    out_specs=pl.BlockSpec((_MMB // 8, 128), lambda i: (i, 0)),
    out_shape=jax.ShapeDtypeStruct((_NR, 128), jnp.float32),
)


def _mid_body(yp_ref, sp_ref, b1p_ref, w2blk_ref, zp_ref):
    h = (yp_ref[...] + sp_ref[:_NR, :] + sp_ref[_NPR : _NPR + _NR, :]
         + b1p_ref[...])
    h = jnp.maximum(h, 0.0)
    zp_ref[...] = jnp.dot(h, w2blk_ref[...], preferred_element_type=jnp.float32)


_mid = pl.pallas_call(
    _mid_body,
    out_shape=jax.ShapeDtypeStruct((_NR, 128), jnp.float32),
)


def _final_body(zp_ref, sp_ref, b2p_ref, batcht_ref, wfc_ref, bfc_ref, o_ref):
    h2 = (zp_ref[...] + sp_ref[:_NR, :] + sp_ref[_NPR : _NPR + _NR, :]
          + b2p_ref[...])
    h2 = jnp.maximum(h2, 0.0)
    gid = lax.broadcasted_iota(jnp.int32, (_G, _NR), 0)
    sums = jnp.zeros((_G, _H), jnp.float32)
    counts = jnp.zeros((_G, 1), jnp.float32)
    for a in range(8):
        mask = jnp.where(batcht_ref[a : a + 1, :] == gid, 1.0, 0.0)
        sums = sums + jnp.dot(mask, h2[:, 16 * a : 16 * (a + 1)],
                              preferred_element_type=jnp.float32)
        counts = counts + jnp.sum(mask, axis=1, keepdims=True)
    pooled = sums / jnp.maximum(counts, 1.0)
    o_ref[...] = (
        jnp.dot(pooled, wfc_ref[...], preferred_element_type=jnp.float32)
        + bfc_ref[...]
    )


_final = pl.pallas_call(
    _final_body,
    out_shape=jax.ShapeDtypeStruct((_G, _C), jnp.float32),
)


def kernel(x, edge_index, batch, W1, b1, W2, b2, Wfc, bfc):
    w2blk = jnp.kron(jnp.eye(8, dtype=jnp.float32), W2)
    b1p = jnp.tile(b1, 8).reshape(1, 128)
    b2p = jnp.tile(b2, 8).reshape(1, 128)
    batcht = batch.reshape(_NR, 8).T
    yp = _mm1(x, W1)
    s1 = _edge_segsum(yp.reshape(_N, _H), edge_index)
    zp = _mid(yp, s1.reshape(2 * _NPR, 128), b1p, w2blk)
    s2 = _edge_segsum(zp.reshape(_N, _H), edge_index)
    return _final(zp, s2.reshape(2 * _NPR, 128), b2p, batcht, Wfc,
                  bfc.reshape(1, _C))
